# baseline (speedup 1.0000x reference)
import jax
import jax.numpy as jnp
from jax.experimental import pallas as pl
from jax.experimental.pallas import tpu as pltpu

_TB = 4096


def _mlp_kernel(x_ref, w0_ref, b0_ref, w1_ref, b1_ref, w2_ref, b2_ref,
                out_ref, w1b_ref, w2b_ref):
    @pl.when(pl.program_id(0) == 0)
    def _():
        w1b_ref[...] = w1_ref[...].astype(jnp.bfloat16)
        w2b_ref[...] = w2_ref[...].astype(jnp.bfloat16)

    mc = 1024
    for s in range(_TB // mc):
        r = slice(s * mc, (s + 1) * mc)
        h = jnp.dot(x_ref[r, :], w0_ref[...],
                    preferred_element_type=jnp.float32)
        h = jnp.maximum(h + b0_ref[...], 0.0).astype(jnp.bfloat16)
        h = jnp.dot(h, w1b_ref[...], preferred_element_type=jnp.float32)
        h = jnp.maximum(h + b1_ref[...], 0.0).astype(jnp.bfloat16)
        h = jnp.dot(h, w2b_ref[...], preferred_element_type=jnp.float32)
        out_ref[r, :] = jnp.maximum(h + b2_ref[...], 0.0)


def kernel(x, w0, b0, w1, b1, w2, b2):
    x = jax.lax.stop_gradient(x)
    B, Din = x.shape
    d0, d1, d2 = w0.shape[1], w1.shape[1], w2.shape[1]

    n_tiles = pl.cdiv(B, _TB)
    flops = 2 * B * (Din * d0 + d0 * d1 + d1 * d2)
    bytes_accessed = (B * Din * 4 + B * d2 * 4
                      + (w0.size + w1.size + w2.size) * 4
                      + (d0 + d1 + d2) * 4)
    return pl.pallas_call(
        _mlp_kernel,
        out_shape=jax.ShapeDtypeStruct((B, d2), x.dtype),
        grid=(n_tiles,),
        in_specs=[
            pl.BlockSpec((_TB, Din), lambda i: (i, 0)),
            pl.BlockSpec(w0.shape, lambda i: (0, 0)),
            pl.BlockSpec(b0.shape, lambda i: (0, 0)),
            pl.BlockSpec(w1.shape, lambda i: (0, 0)),
            pl.BlockSpec(b1.shape, lambda i: (0, 0)),
            pl.BlockSpec(w2.shape, lambda i: (0, 0)),
            pl.BlockSpec(b2.shape, lambda i: (0, 0)),
        ],
        out_specs=pl.BlockSpec((_TB, d2), lambda i: (i, 0)),
        scratch_shapes=[
            pltpu.VMEM(w1.shape, jnp.bfloat16),
            pltpu.VMEM(w2.shape, jnp.bfloat16),
        ],
        cost_estimate=pl.CostEstimate(
            flops=flops, transcendentals=0, bytes_accessed=bytes_accessed),
        compiler_params=pltpu.CompilerParams(
            dimension_semantics=("arbitrary",),
            vmem_limit_bytes=64 << 20),
    )(x, w0, b0, w1, b1, w2, b2)


# speedup vs baseline: 1.2310x; 1.2310x over previous
import jax
import jax.numpy as jnp
from jax.experimental import pallas as pl
from jax.experimental.pallas import tpu as pltpu

_TB = 4096


def _mlp_kernel(x_ref, w0_ref, b0_ref, w1_ref, b1_ref, w2_ref, b2_ref,
                out_ref, w1b_ref, w2b_ref):
    @pl.when(pl.program_id(0) == 0)
    def _():
        w1b_ref[...] = w1_ref[...].astype(jnp.bfloat16)
        w2b_ref[...] = w2_ref[...].astype(jnp.bfloat16)

    h = jnp.dot(x_ref[...], w0_ref[...], preferred_element_type=jnp.float32)
    h = jnp.maximum(h + b0_ref[...], 0.0).astype(jnp.bfloat16)
    h = jnp.dot(h, w1b_ref[...], preferred_element_type=jnp.float32)
    h = jnp.maximum(h + b1_ref[...], 0.0).astype(jnp.bfloat16)
    h = jnp.dot(h, w2b_ref[...], preferred_element_type=jnp.float32)
    out_ref[...] = jnp.maximum(h + b2_ref[...], 0.0)


def kernel(x, w0, b0, w1, b1, w2, b2):
    x = jax.lax.stop_gradient(x)
    B, Din = x.shape
    d0, d1, d2 = w0.shape[1], w1.shape[1], w2.shape[1]

    n_tiles = pl.cdiv(B, _TB)
    flops = 2 * B * (Din * d0 + d0 * d1 + d1 * d2)
    bytes_accessed = (B * Din * 4 + B * d2 * 4
                      + (w0.size + w1.size + w2.size) * 4
                      + (d0 + d1 + d2) * 4)
    return pl.pallas_call(
        _mlp_kernel,
        out_shape=jax.ShapeDtypeStruct((B, d2), x.dtype),
        grid=(n_tiles,),
        in_specs=[
            pl.BlockSpec((_TB, Din), lambda i: (i, 0)),
            pl.BlockSpec(w0.shape, lambda i: (0, 0)),
            pl.BlockSpec(b0.shape, lambda i: (0, 0)),
            pl.BlockSpec(w1.shape, lambda i: (0, 0)),
            pl.BlockSpec(b1.shape, lambda i: (0, 0)),
            pl.BlockSpec(w2.shape, lambda i: (0, 0)),
            pl.BlockSpec(b2.shape, lambda i: (0, 0)),
        ],
        out_specs=pl.BlockSpec((_TB, d2), lambda i: (i, 0)),
        scratch_shapes=[
            pltpu.VMEM(w1.shape, jnp.bfloat16),
            pltpu.VMEM(w2.shape, jnp.bfloat16),
        ],
        cost_estimate=pl.CostEstimate(
            flops=flops, transcendentals=0, bytes_accessed=bytes_accessed),
        compiler_params=pltpu.CompilerParams(
            dimension_semantics=("arbitrary",),
            vmem_limit_bytes=64 << 20),
    )(x, w0, b0, w1, b1, w2, b2)
